# Initial kernel scaffold; baseline (speedup 1.0000x reference)
import jax
import jax.numpy as jnp
from jax.experimental import pallas as pl


def kernel(xs, table):
    raise NotImplementedError("write your pallas kernel here")


# speedup vs baseline: 1.4493x; 1.4493x over previous
import functools

import jax
import jax.numpy as jnp
from jax import lax
from jax.experimental import pallas as pl
from jax.experimental.pallas import tpu as pltpu
from jax.experimental.pallas import tpu_sc as plsc

_NC = 2
_NS = 16
_NW = _NC * _NS
_CHUNK = 512


@functools.partial(jax.jit, static_argnames=("total_b", "dim"))
def _gather_rows(idx, table, total_b, dim):
    b_per_w = total_b // _NW
    n_chunks = b_per_w // _CHUNK
    mesh = plsc.VectorSubcoreMesh(core_axis_name="c", subcore_axis_name="s")

    @functools.partial(
        pl.kernel,
        mesh=mesh,
        out_type=jax.ShapeDtypeStruct((total_b, dim), jnp.float32),
        compiler_params=pltpu.CompilerParams(use_tc_tiling_on_sc=False),
        scratch_types=[
            pltpu.VMEM((b_per_w,), jnp.int32),
            pltpu.VMEM((_CHUNK, dim), jnp.float32),
            pltpu.SemaphoreType.DMA,
        ],
    )
    def k(idx_hbm, table_hbm, out_hbm, idx_v, rows_v, gsem):
        wid = lax.axis_index("s") * _NC + lax.axis_index("c")
        base = wid * b_per_w
        pltpu.sync_copy(idx_hbm.at[pl.ds(base, b_per_w)], idx_v)

        def body(i, carry):
            off = i * _CHUNK
            ck = idx_v.at[pl.ds(off, _CHUNK)]
            pltpu.async_copy(table_hbm.at[ck], rows_v, gsem).wait()
            pltpu.sync_copy(rows_v, out_hbm.at[pl.ds(base + off, _CHUNK)])
            return carry

        lax.fori_loop(0, n_chunks, body, 0)

    return k(idx, table)


def kernel(xs, table):
    b, t = xs.shape
    dim = table.shape[1]
    idx = xs.reshape(-1).astype(jnp.int32)
    out = _gather_rows(idx, table, total_b=b * t, dim=dim)
    return out.reshape(b, t, dim)


# speedup vs baseline: 1.4990x; 1.0343x over previous
import functools

import jax
import jax.numpy as jnp
from jax import lax
from jax.experimental import pallas as pl
from jax.experimental.pallas import tpu as pltpu
from jax.experimental.pallas import tpu_sc as plsc

_NC = 2
_NS = 16
_NW = _NC * _NS
_CHUNK = 1280


@functools.partial(jax.jit, static_argnames=("total_b", "dim"))
def _gather_rows(idx, table, total_b, dim):
    b_per_w = total_b // _NW
    n_chunks = b_per_w // _CHUNK
    n_pairs = n_chunks // 2
    mesh = plsc.VectorSubcoreMesh(core_axis_name="c", subcore_axis_name="s")

    @functools.partial(
        pl.kernel,
        mesh=mesh,
        out_type=jax.ShapeDtypeStruct((total_b, dim), jnp.float32),
        compiler_params=pltpu.CompilerParams(use_tc_tiling_on_sc=False),
        scratch_types=[
            pltpu.VMEM((b_per_w,), jnp.int32),
            pltpu.VMEM((_CHUNK, dim), jnp.float32),
            pltpu.VMEM((_CHUNK, dim), jnp.float32),
            pltpu.SemaphoreType.DMA,
            pltpu.SemaphoreType.DMA,
            pltpu.SemaphoreType.DMA,
            pltpu.SemaphoreType.DMA,
        ],
    )
    def k(idx_hbm, table_hbm, out_hbm, idx_v, rows0, rows1, gs0, gs1, os0, os1):
        wid = lax.axis_index("s") * _NC + lax.axis_index("c")
        base = wid * b_per_w
        pltpu.sync_copy(idx_hbm.at[pl.ds(base, b_per_w)], idx_v)

        def g_copy(c, buf, sem):
            return pltpu.make_async_copy(
                table_hbm.at[idx_v.at[pl.ds(c * _CHUNK, _CHUNK)]], buf, sem)

        def o_copy(c, buf, sem):
            return pltpu.make_async_copy(
                buf, out_hbm.at[pl.ds(base + c * _CHUNK, _CHUNK)], sem)

        g_copy(0, rows0, gs0).start()

        def body(p, carry):
            ce = 2 * p
            co = ce + 1

            @pl.when(p > 0)
            def _():
                o_copy(co - 2, rows1, os1).wait()

            g_copy(co, rows1, gs1).start()
            g_copy(ce, rows0, gs0).wait()
            o_copy(ce, rows0, os0).start()
            g_copy(co, rows1, gs1).wait()
            o_copy(ce, rows0, os0).wait()

            @pl.when(p < n_pairs - 1)
            def _():
                g_copy(ce + 2, rows0, gs0).start()

            o_copy(co, rows1, os1).start()
            return carry

        lax.fori_loop(0, n_pairs, body, 0)
        o_copy(n_chunks - 1, rows1, os1).wait()

    return k(idx, table)


def kernel(xs, table):
    b, t = xs.shape
    dim = table.shape[1]
    idx = xs.reshape(-1).astype(jnp.int32)
    out = _gather_rows(idx, table, total_b=b * t, dim=dim)
    return out.reshape(b, t, dim)


# speedup vs baseline: 1.5001x; 1.0007x over previous
import functools

import jax
import jax.numpy as jnp
from jax import lax
from jax.experimental import pallas as pl
from jax.experimental.pallas import tpu as pltpu
from jax.experimental.pallas import tpu_sc as plsc

_NC = 2
_NS = 16
_NW = _NC * _NS
_R = 4


@functools.partial(jax.jit, static_argnames=("batch", "hist", "dim"))
def _embed(xs, table, batch, hist, dim):
    rows_per_w = batch // _NW
    n_groups = rows_per_w // _R
    n_pairs = n_groups // 2
    mesh = plsc.VectorSubcoreMesh(core_axis_name="c", subcore_axis_name="s")

    @functools.partial(
        pl.kernel,
        mesh=mesh,
        out_type=jax.ShapeDtypeStruct((batch, hist, dim), jnp.float32),
        compiler_params=pltpu.CompilerParams(use_tc_tiling_on_sc=False),
        scratch_types=[
            pltpu.VMEM((rows_per_w, hist), jnp.int32),
            pltpu.VMEM((_R, hist, dim), jnp.float32),
            pltpu.VMEM((_R, hist, dim), jnp.float32),
            pltpu.SemaphoreType.DMA,
            pltpu.SemaphoreType.DMA,
            pltpu.SemaphoreType.DMA,
            pltpu.SemaphoreType.DMA,
        ],
    )
    def k(xs_hbm, table_hbm, out_hbm, idx_v, buf0, buf1, gs0, gs1, os0, os1):
        wid = lax.axis_index("s") * _NC + lax.axis_index("c")
        base = wid * rows_per_w
        pltpu.sync_copy(xs_hbm.at[pl.ds(base, rows_per_w)], idx_v)

        def g_copy(g, j, buf, sem):
            return pltpu.make_async_copy(
                table_hbm.at[idx_v.at[g * _R + j]], buf.at[j], sem)

        def o_copy(g, buf, sem):
            return pltpu.make_async_copy(
                buf, out_hbm.at[pl.ds(base + g * _R, _R)], sem)

        for j in range(_R):
            g_copy(0, j, buf0, gs0).start()

        def body(p, carry):
            ge = 2 * p
            go = ge + 1

            @pl.when(p > 0)
            def _():
                o_copy(go - 2, buf1, os1).wait()

            for j in range(_R):
                g_copy(go, j, buf1, gs1).start()
            for j in range(_R):
                g_copy(ge, j, buf0, gs0).wait()
            o_copy(ge, buf0, os0).start()
            for j in range(_R):
                g_copy(go, j, buf1, gs1).wait()
            o_copy(ge, buf0, os0).wait()

            @pl.when(p < n_pairs - 1)
            def _():
                for j in range(_R):
                    g_copy(ge + 2, j, buf0, gs0).start()

            o_copy(go, buf1, os1).start()
            return carry

        lax.fori_loop(0, n_pairs, body, 0)
        o_copy(n_groups - 1, buf1, os1).wait()

    return k(xs, table)


def kernel(xs, table):
    b, t = xs.shape
    dim = table.shape[1]
    return _embed(xs.astype(jnp.int32), table, batch=b, hist=t, dim=dim)
